# baseline (speedup 1.0000x reference)
import functools

import jax
import jax.numpy as jnp
from jax import lax
from jax.experimental import pallas as pl
from jax.experimental.pallas import tpu as pltpu
from jax.experimental.pallas import tpu_sc as plsc

BATCH = 16384
EMBED = 32
CONTENT = 64
HIDDEN = 64

NC = 2
NS = 16
NW = NC * NS
B_PER_W = BATCH // NW
CHUNK = 128
NCH = B_PER_W // CHUNK

@functools.lru_cache(maxsize=None)
def _make_sc_gather():
    mesh = plsc.VectorSubcoreMesh(core_axis_name="c", subcore_axis_name="s")

    @functools.partial(
        pl.kernel,
        mesh=mesh,
        compiler_params=pltpu.CompilerParams(use_tc_tiling_on_sc=False),
        out_type=[
            jax.ShapeDtypeStruct((BATCH, EMBED), jnp.float32),
            jax.ShapeDtypeStruct((BATCH, EMBED), jnp.float32),
        ],
        scratch_types=[
            pltpu.VMEM((B_PER_W,), jnp.int32),
            pltpu.VMEM((B_PER_W,), jnp.int32),
            pltpu.VMEM((B_PER_W, EMBED), jnp.float32),
            pltpu.VMEM((B_PER_W, EMBED), jnp.float32),
            pltpu.SemaphoreType.DMA,
            pltpu.SemaphoreType.DMA,
        ],
    )
    def _sc_gather(uid_hbm, iid_hbm, ut_hbm, it_hbm, uout_hbm, iout_hbm,
                   uidx_v, iidx_v, urows_v, irows_v, usem, isem):
        wid = lax.axis_index("s") * NC + lax.axis_index("c")
        base = wid * B_PER_W
        pltpu.sync_copy(uid_hbm.at[pl.ds(base, B_PER_W)], uidx_v)
        pltpu.sync_copy(iid_hbm.at[pl.ds(base, B_PER_W)], iidx_v)
        copies = []
        for t in range(NCH):
            sl = pl.ds(t * CHUNK, CHUNK)
            copies.append(pltpu.async_copy(ut_hbm.at[uidx_v.at[sl]], urows_v.at[sl], usem))
            copies.append(pltpu.async_copy(it_hbm.at[iidx_v.at[sl]], irows_v.at[sl], isem))
        for cp in copies:
            cp.wait()
        pltpu.sync_copy(urows_v, uout_hbm.at[pl.ds(base, B_PER_W)])
        pltpu.sync_copy(irows_v, iout_hbm.at[pl.ds(base, B_PER_W)])

    return _sc_gather


BT = 2048


def _mlp_body(u_ref, i_ref, c_ref, w1u_ref, w1i_ref, w1c_ref, b1_ref,
              w2_ref, b2_ref, o_ref):
    h = (jnp.dot(u_ref[...], w1u_ref[...], preferred_element_type=jnp.float32)
         + jnp.dot(i_ref[...], w1i_ref[...], preferred_element_type=jnp.float32)
         + jnp.dot(c_ref[...], w1c_ref[...], preferred_element_type=jnp.float32)
         + b1_ref[...])
    h = jnp.maximum(h, 0.0)
    o_ref[...] = jnp.sum(h * w2_ref[...], axis=1, keepdims=True) + b2_ref[...]


def _mlp(u_emb, i_emb, content, w1u, w1i, w1c, b1, w2, b2):
    grid = (BATCH // BT,)
    return pl.pallas_call(
        _mlp_body,
        grid=grid,
        in_specs=[
            pl.BlockSpec((BT, EMBED), lambda i: (i, 0)),
            pl.BlockSpec((BT, EMBED), lambda i: (i, 0)),
            pl.BlockSpec((BT, CONTENT), lambda i: (i, 0)),
            pl.BlockSpec((EMBED, HIDDEN), lambda i: (0, 0)),
            pl.BlockSpec((EMBED, HIDDEN), lambda i: (0, 0)),
            pl.BlockSpec((CONTENT, HIDDEN), lambda i: (0, 0)),
            pl.BlockSpec((1, HIDDEN), lambda i: (0, 0)),
            pl.BlockSpec((1, HIDDEN), lambda i: (0, 0)),
            pl.BlockSpec((1, 1), lambda i: (0, 0)),
        ],
        out_specs=pl.BlockSpec((BT, 1), lambda i: (i, 0)),
        out_shape=jax.ShapeDtypeStruct((BATCH, 1), jnp.float32),
    )(u_emb, i_emb, content, w1u, w1i, w1c, b1, w2, b2)


def kernel(user_ids, item_ids, content_features, user_table, item_table,
           W1, b1, W2, b2):
    uids = user_ids.astype(jnp.int32)
    iids = item_ids.astype(jnp.int32)
    u_emb, i_emb = _make_sc_gather()(uids, iids, user_table, item_table)
    w1u = W1[:EMBED]
    w1i = W1[EMBED:2 * EMBED]
    w1c = W1[2 * EMBED:]
    out = _mlp(u_emb, i_emb, content_features,
               w1u, w1i, w1c,
               b1.reshape(1, HIDDEN),
               W2.reshape(1, HIDDEN),
               b2.reshape(1, 1))
    return out
